# Initial kernel scaffold; baseline (speedup 1.0000x reference)
import jax
import jax.numpy as jnp
from jax.experimental import pallas as pl


def kernel(x, edge_index, W, a):
    raise NotImplementedError("write your pallas kernel here")


# speedup vs baseline: 67.2986x; 67.2986x over previous
import functools

import jax
import jax.numpy as jnp
from jax import lax
from jax.experimental import pallas as pl
from jax.experimental.pallas import tpu as pltpu
from jax.experimental.pallas import tpu_sc as plsc

N = 4096
E = 131072
IN_F = 256
OUT_F = 64
H = 4
NW = 32
EPW = E // NW
ROWB = 512
NRB = N // ROWB


def _mm_body(x_ref, wr_ref, ac_ref, xt_ref, proj_ref, cmax_ref):
    r = pl.program_id(0)
    xt = jnp.dot(x_ref[...], wr_ref[...], preferred_element_type=jnp.float32)
    xt_ref[...] = xt
    p = jnp.dot(xt, ac_ref[...], preferred_element_type=jnp.float32)
    proj_ref[...] = p
    pm = jnp.max(p, axis=0, keepdims=True)

    @pl.when(r == 0)
    def _():
        cmax_ref[...] = pm

    @pl.when(r > 0)
    def _():
        cmax_ref[...] = jnp.maximum(cmax_ref[...], pm)


_mm_call = pl.pallas_call(
    _mm_body,
    grid=(NRB,),
    in_specs=[
        pl.BlockSpec((ROWB, IN_F), lambda r: (r, 0)),
        pl.BlockSpec((IN_F, H * OUT_F), lambda r: (0, 0)),
        pl.BlockSpec((IN_F, 2 * H), lambda r: (0, 0)),
    ],
    out_specs=[
        pl.BlockSpec((ROWB, H * OUT_F), lambda r: (r, 0)),
        pl.BlockSpec((ROWB, 2 * H), lambda r: (r, 0)),
        pl.BlockSpec((1, 2 * H), lambda r: (0, 0)),
    ],
    out_shape=[
        jax.ShapeDtypeStruct((N, H * OUT_F), jnp.float32),
        jax.ShapeDtypeStruct((N, 2 * H), jnp.float32),
        jax.ShapeDtypeStruct((1, 2 * H), jnp.float32),
    ],
)


def _edge_kernel_body(proj_hbm, src_hbm, dst_hbm, c_hbm,
                      den_out, self_out,
                      proj_v, src_v, dst_v, c_v, den_v, self_v):
    wid = lax.axis_index("s") * 2 + lax.axis_index("c")
    base = wid * EPW
    pltpu.sync_copy(proj_hbm, proj_v)
    pltpu.sync_copy(src_hbm.at[pl.ds(base, EPW)], src_v)
    pltpu.sync_copy(dst_hbm.at[pl.ds(base, EPW)], dst_v)
    pltpu.sync_copy(c_hbm, c_v)

    zeros16 = jnp.zeros((16,), jnp.float32)

    def zero_den(i, carry):
        den_v[pl.ds(i * 16, 16)] = zeros16
        return carry

    lax.fori_loop(0, N * H // 16, zero_den, 0)

    def zero_self(i, carry):
        self_v[pl.ds(i * 16, 16)] = zeros16
        return carry

    lax.fori_loop(0, N // 16, zero_self, 0)

    cvecs = [c_v[pl.ds(h * 16, 16)] for h in range(H)]
    ones16 = jnp.ones((16,), jnp.float32)

    def body(i, carry):
        s16 = src_v[pl.ds(i * 16, 16)]
        d16 = dst_v[pl.ds(i * 16, 16)]
        plsc.addupdate_scatter(
            self_v, [d16], jnp.where(s16 == d16, ones16, zeros16)
        )
        s8 = s16 * (2 * H)
        d8 = d16 * (2 * H)
        dh = d16 * H
        for h in range(H):
            gs = plsc.load_gather(proj_v, [s8 + h])
            gd = plsc.load_gather(proj_v, [d8 + (H + h)])
            t = gs + gd
            e = jnp.maximum(t, t * 0.2)
            v = jnp.exp(e - cvecs[h])
            plsc.addupdate_scatter(den_v, [dh + h], v)
        return carry

    lax.fori_loop(0, EPW // 16, body, 0)

    pltpu.sync_copy(den_v, den_out.at[wid])
    pltpu.sync_copy(self_v, self_out.at[wid])


@functools.lru_cache(maxsize=1)
def _make_edge_kernel():
    mesh = plsc.VectorSubcoreMesh(core_axis_name="c", subcore_axis_name="s")
    return pl.kernel(
        _edge_kernel_body,
        out_type=[
            jax.ShapeDtypeStruct((NW, N * H), jnp.float32),
            jax.ShapeDtypeStruct((NW, N), jnp.float32),
        ],
        mesh=mesh,
        compiler_params=pltpu.CompilerParams(needs_layout_passes=False),
        scratch_types=[
            pltpu.VMEM((N * 2 * H,), jnp.float32),
            pltpu.VMEM((EPW,), jnp.int32),
            pltpu.VMEM((EPW,), jnp.int32),
            pltpu.VMEM((H * 16,), jnp.float32),
            pltpu.VMEM((N * H,), jnp.float32),
            pltpu.VMEM((N,), jnp.float32),
        ],
    )


def _fin_body(xt_ref, den_ref, self_ref, proj_ref, c_ref, exp_ref, out_ref):
    den = jnp.sum(den_ref[...], axis=0)
    cnt = jnp.sum(self_ref[...], axis=0)
    t = proj_ref[:, :H] + proj_ref[:, H:]
    e = jnp.maximum(t, t * 0.2)
    v = jnp.exp(e - c_ref[...])
    scale = jnp.where(cnt > 0.0, v / (den + 1e-16), 0.0)
    wide = jnp.dot(scale, exp_ref[...], preferred_element_type=jnp.float32)
    out_ref[...] = xt_ref[...] * wide


_fin_call = pl.pallas_call(
    _fin_body,
    grid=(NRB,),
    in_specs=[
        pl.BlockSpec((ROWB, H * OUT_F), lambda r: (r, 0)),
        pl.BlockSpec((NW, ROWB, H), lambda r: (0, r, 0)),
        pl.BlockSpec((NW, ROWB, 1), lambda r: (0, r, 0)),
        pl.BlockSpec((ROWB, 2 * H), lambda r: (r, 0)),
        pl.BlockSpec((1, H), lambda r: (0, 0)),
        pl.BlockSpec((H, H * OUT_F), lambda r: (0, 0)),
    ],
    out_specs=pl.BlockSpec((ROWB, H * OUT_F), lambda r: (r, 0)),
    out_shape=jax.ShapeDtypeStruct((N, H * OUT_F), jnp.float32),
)


def kernel(x, edge_index, W, a):
    x = x.astype(jnp.float32)
    Wr = jnp.transpose(W, (1, 0, 2)).reshape(IN_F, H * OUT_F).astype(jnp.float32)
    a2 = a[..., 0].astype(jnp.float32)
    eye = jnp.eye(H, dtype=jnp.float32)
    ac_s = (a2[:, :OUT_F][:, :, None] * eye[:, None, :]).reshape(H * OUT_F, H)
    ac_d = (a2[:, OUT_F:][:, :, None] * eye[:, None, :]).reshape(H * OUT_F, H)
    Ac = jnp.concatenate([ac_s, ac_d], axis=1)

    xt, proj, cmax = _mm_call(x, Wr, Ac)

    mub = cmax[0, :H] + cmax[0, H:]
    cbound = jnp.maximum(mub, 0.2 * mub)
    c_lanes = jnp.broadcast_to(cbound[:, None], (H, 16)).reshape(H * 16)

    src = edge_index[0].astype(jnp.int32)
    dst = edge_index[1].astype(jnp.int32)
    den_parts, self_parts = _make_edge_kernel()(proj.reshape(-1), src, dst, c_lanes)

    expander = jnp.repeat(jnp.eye(H, dtype=jnp.float32), OUT_F, axis=1)
    out = _fin_call(
        xt,
        den_parts.reshape(NW, N, H),
        self_parts.reshape(NW, N, 1),
        proj,
        cbound.reshape(1, H),
        expander,
    )
    return out


# speedup vs baseline: 182.8976x; 2.7177x over previous
import functools

import jax
import jax.numpy as jnp
from jax import lax
from jax.experimental import pallas as pl
from jax.experimental.pallas import tpu as pltpu
from jax.experimental.pallas import tpu_sc as plsc

N = 4096
E = 131072
IN_F = 256
OUT_F = 64
H = 4
NW = 32
EPW = E // NW
ROWB = 512
NRB = N // ROWB


def _mm_body(x_ref, wr_ref, ac_ref, xt_ref, projt_ref, cmax_ref):
    r = pl.program_id(0)
    xt = jnp.dot(x_ref[...], wr_ref[...], preferred_element_type=jnp.float32)
    xt_ref[...] = xt
    p = jnp.dot(xt, ac_ref[...], preferred_element_type=jnp.float32)
    projt_ref[...] = p.T
    pm = jnp.max(p, axis=0, keepdims=True)

    @pl.when(r == 0)
    def _():
        cmax_ref[...] = pm

    @pl.when(r > 0)
    def _():
        cmax_ref[...] = jnp.maximum(cmax_ref[...], pm)


_mm_call = pl.pallas_call(
    _mm_body,
    grid=(NRB,),
    in_specs=[
        pl.BlockSpec((ROWB, IN_F), lambda r: (r, 0)),
        pl.BlockSpec((IN_F, H * OUT_F), lambda r: (0, 0)),
        pl.BlockSpec((IN_F, 2 * H), lambda r: (0, 0)),
    ],
    out_specs=[
        pl.BlockSpec((ROWB, H * OUT_F), lambda r: (r, 0)),
        pl.BlockSpec((2 * H, ROWB), lambda r: (0, r)),
        pl.BlockSpec((1, 2 * H), lambda r: (0, 0)),
    ],
    out_shape=[
        jax.ShapeDtypeStruct((N, H * OUT_F), jnp.float32),
        jax.ShapeDtypeStruct((2 * H, N), jnp.float32),
        jax.ShapeDtypeStruct((1, 2 * H), jnp.float32),
    ],
)


def _edge_kernel_body(projt_hbm, src_hbm, dst_hbm, c_hbm,
                      d0_out, d1_out, d2_out, d3_out, self_out,
                      projt_v, src_v, dst_v, c_v,
                      d0_v, d1_v, d2_v, d3_v, self_v):
    wid = lax.axis_index("s") * 2 + lax.axis_index("c")
    base = wid * EPW
    pltpu.sync_copy(projt_hbm, projt_v)
    pltpu.sync_copy(src_hbm.at[pl.ds(base, EPW)], src_v)
    pltpu.sync_copy(dst_hbm.at[pl.ds(base, EPW)], dst_v)
    pltpu.sync_copy(c_hbm, c_v)

    den_refs = [d0_v, d1_v, d2_v, d3_v]
    zeros16 = jnp.zeros((16,), jnp.float32)
    ones16 = jnp.ones((16,), jnp.float32)

    def zero_body(i, carry):
        sl = pl.ds(i * 16, 16)
        d0_v[sl] = zeros16
        d1_v[sl] = zeros16
        d2_v[sl] = zeros16
        d3_v[sl] = zeros16
        self_v[sl] = zeros16
        return carry

    lax.fori_loop(0, N // 16, zero_body, 0)

    cvecs = [c_v[pl.ds(h * 16, 16)] for h in range(H)]
    rows = [jnp.full((16,), h, jnp.int32) for h in range(2 * H)]

    def body(i, carry):
        sl = pl.ds(i * 16, 16)
        s16 = src_v[sl]
        d16 = dst_v[sl]
        plsc.addupdate_scatter(
            self_v, [d16], jnp.where(s16 == d16, ones16, zeros16)
        )
        for h in range(H):
            gs = plsc.load_gather(projt_v, [rows[h], s16])
            gd = plsc.load_gather(projt_v, [rows[H + h], d16])
            t = gs + gd
            e = jnp.maximum(t, t * 0.2)
            v = jnp.exp(e - cvecs[h])
            plsc.addupdate_scatter(den_refs[h], [d16], v)
        return carry

    lax.fori_loop(0, EPW // 16, body, 0)

    pltpu.sync_copy(d0_v, d0_out.at[wid])
    pltpu.sync_copy(d1_v, d1_out.at[wid])
    pltpu.sync_copy(d2_v, d2_out.at[wid])
    pltpu.sync_copy(d3_v, d3_out.at[wid])
    pltpu.sync_copy(self_v, self_out.at[wid])


@functools.lru_cache(maxsize=1)
def _make_edge_kernel():
    mesh = plsc.VectorSubcoreMesh(core_axis_name="c", subcore_axis_name="s")
    return pl.kernel(
        _edge_kernel_body,
        out_type=[jax.ShapeDtypeStruct((NW, N), jnp.float32)] * 5,
        mesh=mesh,
        compiler_params=pltpu.CompilerParams(needs_layout_passes=False),
        scratch_types=[
            pltpu.VMEM((2 * H, N), jnp.float32),
            pltpu.VMEM((EPW,), jnp.int32),
            pltpu.VMEM((EPW,), jnp.int32),
            pltpu.VMEM((H * 16,), jnp.float32),
            pltpu.VMEM((N,), jnp.float32),
            pltpu.VMEM((N,), jnp.float32),
            pltpu.VMEM((N,), jnp.float32),
            pltpu.VMEM((N,), jnp.float32),
            pltpu.VMEM((N,), jnp.float32),
        ],
    )


def _fin_body(xt_ref, d0_ref, d1_ref, d2_ref, d3_ref, self_ref,
              projt_ref, c_ref, exp_ref, out_ref):
    dens = [jnp.sum(d[...], axis=0, keepdims=True)
            for d in (d0_ref, d1_ref, d2_ref, d3_ref)]
    den = jnp.concatenate(dens, axis=0)
    cnt = jnp.sum(self_ref[...], axis=0, keepdims=True)
    t = projt_ref[:H, :] + projt_ref[H:, :]
    e = jnp.maximum(t, t * 0.2)
    v = jnp.exp(e - c_ref[...])
    scale = jnp.where(cnt > 0.0, v / (den + 1e-16), 0.0)
    wide = lax.dot_general(
        scale, exp_ref[...],
        dimension_numbers=(((0,), (0,)), ((), ())),
        preferred_element_type=jnp.float32,
    )
    out_ref[...] = xt_ref[...] * wide


_fin_call = pl.pallas_call(
    _fin_body,
    grid=(NRB,),
    in_specs=[
        pl.BlockSpec((ROWB, H * OUT_F), lambda r: (r, 0)),
        pl.BlockSpec((NW, ROWB), lambda r: (0, r)),
        pl.BlockSpec((NW, ROWB), lambda r: (0, r)),
        pl.BlockSpec((NW, ROWB), lambda r: (0, r)),
        pl.BlockSpec((NW, ROWB), lambda r: (0, r)),
        pl.BlockSpec((NW, ROWB), lambda r: (0, r)),
        pl.BlockSpec((2 * H, ROWB), lambda r: (0, r)),
        pl.BlockSpec((H, 1), lambda r: (0, 0)),
        pl.BlockSpec((H, H * OUT_F), lambda r: (0, 0)),
    ],
    out_specs=pl.BlockSpec((ROWB, H * OUT_F), lambda r: (r, 0)),
    out_shape=jax.ShapeDtypeStruct((N, H * OUT_F), jnp.float32),
)


def kernel(x, edge_index, W, a):
    x = x.astype(jnp.float32)
    Wr = jnp.transpose(W, (1, 0, 2)).reshape(IN_F, H * OUT_F).astype(jnp.float32)
    a2 = a[..., 0].astype(jnp.float32)
    eye = jnp.eye(H, dtype=jnp.float32)
    ac_s = (a2[:, :OUT_F][:, :, None] * eye[:, None, :]).reshape(H * OUT_F, H)
    ac_d = (a2[:, OUT_F:][:, :, None] * eye[:, None, :]).reshape(H * OUT_F, H)
    Ac = jnp.concatenate([ac_s, ac_d], axis=1)

    xt, projt, cmax = _mm_call(x, Wr, Ac)

    mub = cmax[0, :H] + cmax[0, H:]
    cbound = jnp.maximum(mub, 0.2 * mub)
    c_lanes = jnp.broadcast_to(cbound[:, None], (H, 16)).reshape(H * 16)

    src = edge_index[0].astype(jnp.int32)
    dst = edge_index[1].astype(jnp.int32)
    d0, d1, d2, d3, self_parts = _make_edge_kernel()(projt, src, dst, c_lanes)

    expander = jnp.repeat(jnp.eye(H, dtype=jnp.float32), OUT_F, axis=1)
    out = _fin_call(
        xt, d0, d1, d2, d3, self_parts,
        projt, cbound.reshape(H, 1), expander,
    )
    return out


# speedup vs baseline: 227.5837x; 1.2443x over previous
import functools

import jax
import jax.numpy as jnp
from jax import lax
from jax.experimental import pallas as pl
from jax.experimental.pallas import tpu as pltpu
from jax.experimental.pallas import tpu_sc as plsc

N = 4096
E = 131072
IN_F = 256
OUT_F = 64
H = 4
NW = 32
EPW = E // NW
ROWB = 512
NRB = N // ROWB


def _mm_body(x_ref, wr_ref, ac_ref, xt_ref, projt_ref, cmax_ref):
    r = pl.program_id(0)
    xt = jnp.dot(x_ref[...], wr_ref[...], preferred_element_type=jnp.float32)
    xt_ref[...] = xt
    p = jnp.dot(xt, ac_ref[...], preferred_element_type=jnp.float32)
    projt_ref[...] = p.T
    pm = jnp.max(p, axis=0, keepdims=True)

    @pl.when(r == 0)
    def _():
        cmax_ref[...] = pm

    @pl.when(r > 0)
    def _():
        cmax_ref[...] = jnp.maximum(cmax_ref[...], pm)


_mm_call = pl.pallas_call(
    _mm_body,
    grid=(NRB,),
    in_specs=[
        pl.BlockSpec((ROWB, IN_F), lambda r: (r, 0)),
        pl.BlockSpec((IN_F, H * OUT_F), lambda r: (0, 0)),
        pl.BlockSpec((IN_F, 2 * H), lambda r: (0, 0)),
    ],
    out_specs=[
        pl.BlockSpec((ROWB, H * OUT_F), lambda r: (r, 0)),
        pl.BlockSpec((2 * H, ROWB), lambda r: (0, r)),
        pl.BlockSpec((1, 2 * H), lambda r: (0, 0)),
    ],
    out_shape=[
        jax.ShapeDtypeStruct((N, H * OUT_F), jnp.float32),
        jax.ShapeDtypeStruct((2 * H, N), jnp.float32),
        jax.ShapeDtypeStruct((1, 2 * H), jnp.float32),
    ],
)


def _edge_kernel_body(projt_hbm, src_hbm, dst_hbm, c_hbm,
                      d0_out, d1_out, d2_out, d3_out, self_out,
                      projt_v, src_v, dst_v, c_v,
                      d0_v, d1_v, d2_v, d3_v, self_v):
    wid = lax.axis_index("s") * 2 + lax.axis_index("c")
    base = wid * EPW
    pltpu.sync_copy(projt_hbm, projt_v)
    pltpu.sync_copy(src_hbm.at[pl.ds(base, EPW)], src_v)
    pltpu.sync_copy(dst_hbm.at[pl.ds(base, EPW)], dst_v)
    pltpu.sync_copy(c_hbm, c_v)

    den_refs = [d0_v, d1_v, d2_v, d3_v]
    zeros16 = jnp.zeros((16,), jnp.float32)
    ones16 = jnp.ones((16,), jnp.float32)

    @plsc.parallel_loop(0, N // 16, unroll=8)
    def _(i):
        sl = pl.ds(i * 16, 16)
        d0_v[sl] = zeros16
        d1_v[sl] = zeros16
        d2_v[sl] = zeros16
        d3_v[sl] = zeros16
        self_v[sl] = zeros16

    cvecs = [c_v[pl.ds(h * 16, 16)] for h in range(H)]
    rows = [jnp.full((16,), h, jnp.int32) for h in range(2 * H)]

    @plsc.parallel_loop(0, EPW // 16, unroll=4)
    def _(i):
        sl = pl.ds(i * 16, 16)
        s16 = src_v[sl]
        d16 = dst_v[sl]
        plsc.addupdate_scatter(
            self_v, [d16], jnp.where(s16 == d16, ones16, zeros16)
        )
        for h in range(H):
            gs = plsc.load_gather(projt_v, [rows[h], s16])
            gd = plsc.load_gather(projt_v, [rows[H + h], d16])
            t = gs + gd
            e = jnp.maximum(t, t * 0.2)
            v = jnp.exp(e - cvecs[h])
            plsc.addupdate_scatter(den_refs[h], [d16], v)

    pltpu.sync_copy(d0_v, d0_out.at[wid])
    pltpu.sync_copy(d1_v, d1_out.at[wid])
    pltpu.sync_copy(d2_v, d2_out.at[wid])
    pltpu.sync_copy(d3_v, d3_out.at[wid])
    pltpu.sync_copy(self_v, self_out.at[wid])


@functools.lru_cache(maxsize=1)
def _make_edge_kernel():
    mesh = plsc.VectorSubcoreMesh(core_axis_name="c", subcore_axis_name="s")
    return pl.kernel(
        _edge_kernel_body,
        out_type=[jax.ShapeDtypeStruct((NW, N), jnp.float32)] * 5,
        mesh=mesh,
        compiler_params=pltpu.CompilerParams(needs_layout_passes=False),
        scratch_types=[
            pltpu.VMEM((2 * H, N), jnp.float32),
            pltpu.VMEM((EPW,), jnp.int32),
            pltpu.VMEM((EPW,), jnp.int32),
            pltpu.VMEM((H * 16,), jnp.float32),
            pltpu.VMEM((N,), jnp.float32),
            pltpu.VMEM((N,), jnp.float32),
            pltpu.VMEM((N,), jnp.float32),
            pltpu.VMEM((N,), jnp.float32),
            pltpu.VMEM((N,), jnp.float32),
        ],
    )


def _fin_body(xt_ref, d0_ref, d1_ref, d2_ref, d3_ref, self_ref,
              projt_ref, c_ref, exp_ref, out_ref):
    dens = [jnp.sum(d[...], axis=0, keepdims=True)
            for d in (d0_ref, d1_ref, d2_ref, d3_ref)]
    den = jnp.concatenate(dens, axis=0)
    cnt = jnp.sum(self_ref[...], axis=0, keepdims=True)
    t = projt_ref[:H, :] + projt_ref[H:, :]
    e = jnp.maximum(t, t * 0.2)
    v = jnp.exp(e - c_ref[...])
    scale = jnp.where(cnt > 0.0, v / (den + 1e-16), 0.0)
    wide = lax.dot_general(
        scale, exp_ref[...],
        dimension_numbers=(((0,), (0,)), ((), ())),
        preferred_element_type=jnp.float32,
    )
    out_ref[...] = xt_ref[...] * wide


_fin_call = pl.pallas_call(
    _fin_body,
    grid=(NRB,),
    in_specs=[
        pl.BlockSpec((ROWB, H * OUT_F), lambda r: (r, 0)),
        pl.BlockSpec((NW, ROWB), lambda r: (0, r)),
        pl.BlockSpec((NW, ROWB), lambda r: (0, r)),
        pl.BlockSpec((NW, ROWB), lambda r: (0, r)),
        pl.BlockSpec((NW, ROWB), lambda r: (0, r)),
        pl.BlockSpec((NW, ROWB), lambda r: (0, r)),
        pl.BlockSpec((2 * H, ROWB), lambda r: (0, r)),
        pl.BlockSpec((H, 1), lambda r: (0, 0)),
        pl.BlockSpec((H, H * OUT_F), lambda r: (0, 0)),
    ],
    out_specs=pl.BlockSpec((ROWB, H * OUT_F), lambda r: (r, 0)),
    out_shape=jax.ShapeDtypeStruct((N, H * OUT_F), jnp.float32),
)


def kernel(x, edge_index, W, a):
    x = x.astype(jnp.float32)
    Wr = jnp.transpose(W, (1, 0, 2)).reshape(IN_F, H * OUT_F).astype(jnp.float32)
    a2 = a[..., 0].astype(jnp.float32)
    eye = jnp.eye(H, dtype=jnp.float32)
    ac_s = (a2[:, :OUT_F][:, :, None] * eye[:, None, :]).reshape(H * OUT_F, H)
    ac_d = (a2[:, OUT_F:][:, :, None] * eye[:, None, :]).reshape(H * OUT_F, H)
    Ac = jnp.concatenate([ac_s, ac_d], axis=1)

    xt, projt, cmax = _mm_call(x, Wr, Ac)

    mub = cmax[0, :H] + cmax[0, H:]
    cbound = jnp.maximum(mub, 0.2 * mub)
    c_lanes = jnp.broadcast_to(cbound[:, None], (H, 16)).reshape(H * 16)

    src = edge_index[0].astype(jnp.int32)
    dst = edge_index[1].astype(jnp.int32)
    d0, d1, d2, d3, self_parts = _make_edge_kernel()(projt, src, dst, c_lanes)

    expander = jnp.repeat(jnp.eye(H, dtype=jnp.float32), OUT_F, axis=1)
    out = _fin_call(
        xt, d0, d1, d2, d3, self_parts,
        projt, cbound.reshape(H, 1), expander,
    )
    return out


# speedup vs baseline: 229.6305x; 1.0090x over previous
import functools

import jax
import jax.numpy as jnp
from jax import lax
from jax.experimental import pallas as pl
from jax.experimental.pallas import tpu as pltpu
from jax.experimental.pallas import tpu_sc as plsc

N = 4096
E = 131072
IN_F = 256
OUT_F = 64
H = 4
NW = 32
EPW = E // NW
ROWB = 512
NRB = N // ROWB


def _proj_body(x_ref, wr_ref, ac_ref, projt_ref, cmax_ref):
    r = pl.program_id(0)
    wc = jnp.dot(wr_ref[...], ac_ref[...], preferred_element_type=jnp.float32)
    p = jnp.dot(x_ref[...], wc, preferred_element_type=jnp.float32)
    projt_ref[...] = p.T
    pm = jnp.max(p, axis=0, keepdims=True)

    @pl.when(r == 0)
    def _():
        cmax_ref[...] = pm

    @pl.when(r > 0)
    def _():
        cmax_ref[...] = jnp.maximum(cmax_ref[...], pm)


_proj_call = pl.pallas_call(
    _proj_body,
    grid=(NRB,),
    in_specs=[
        pl.BlockSpec((ROWB, IN_F), lambda r: (r, 0)),
        pl.BlockSpec((IN_F, H * OUT_F), lambda r: (0, 0)),
        pl.BlockSpec((IN_F, 2 * H), lambda r: (0, 0)),
    ],
    out_specs=[
        pl.BlockSpec((2 * H, ROWB), lambda r: (0, r)),
        pl.BlockSpec((1, 2 * H), lambda r: (0, 0)),
    ],
    out_shape=[
        jax.ShapeDtypeStruct((2 * H, N), jnp.float32),
        jax.ShapeDtypeStruct((1, 2 * H), jnp.float32),
    ],
)


def _edge_kernel_body(projt_hbm, edge_hbm, c_hbm,
                      d0_out, d1_out, d2_out, d3_out, self_out,
                      projt_v, src_v, dst_v, c_v,
                      d0_v, d1_v, d2_v, d3_v, self_v):
    wid = lax.axis_index("s") * 2 + lax.axis_index("c")
    base = wid * EPW
    pltpu.sync_copy(projt_hbm, projt_v)
    pltpu.sync_copy(edge_hbm.at[0, pl.ds(base, EPW)], src_v)
    pltpu.sync_copy(edge_hbm.at[1, pl.ds(base, EPW)], dst_v)
    pltpu.sync_copy(c_hbm, c_v)

    den_refs = [d0_v, d1_v, d2_v, d3_v]
    zeros16 = jnp.zeros((16,), jnp.float32)
    ones16 = jnp.ones((16,), jnp.float32)

    @plsc.parallel_loop(0, N // 16, unroll=8)
    def _(i):
        sl = pl.ds(i * 16, 16)
        d0_v[sl] = zeros16
        d1_v[sl] = zeros16
        d2_v[sl] = zeros16
        d3_v[sl] = zeros16
        self_v[sl] = zeros16

    cvecs = [c_v[pl.ds(h * 16, 16)] for h in range(H)]
    rows = [jnp.full((16,), h, jnp.int32) for h in range(2 * H)]

    @plsc.parallel_loop(0, EPW // 16, unroll=8)
    def _(i):
        sl = pl.ds(i * 16, 16)
        s16 = src_v[sl]
        d16 = dst_v[sl]
        plsc.addupdate_scatter(
            self_v, [d16], jnp.where(s16 == d16, ones16, zeros16)
        )
        for h in range(H):
            gs = plsc.load_gather(projt_v, [rows[h], s16])
            gd = plsc.load_gather(projt_v, [rows[H + h], d16])
            t = gs + gd
            e = jnp.maximum(t, t * 0.2)
            v = jnp.exp(e - cvecs[h])
            plsc.addupdate_scatter(den_refs[h], [d16], v)

    pltpu.sync_copy(d0_v, d0_out.at[wid])
    pltpu.sync_copy(d1_v, d1_out.at[wid])
    pltpu.sync_copy(d2_v, d2_out.at[wid])
    pltpu.sync_copy(d3_v, d3_out.at[wid])
    pltpu.sync_copy(self_v, self_out.at[wid])


@functools.lru_cache(maxsize=1)
def _make_edge_kernel():
    mesh = plsc.VectorSubcoreMesh(core_axis_name="c", subcore_axis_name="s")
    return pl.kernel(
        _edge_kernel_body,
        out_type=[jax.ShapeDtypeStruct((NW, N), jnp.float32)] * 5,
        mesh=mesh,
        compiler_params=pltpu.CompilerParams(needs_layout_passes=False),
        scratch_types=[
            pltpu.VMEM((2 * H, N), jnp.float32),
            pltpu.VMEM((EPW,), jnp.int32),
            pltpu.VMEM((EPW,), jnp.int32),
            pltpu.VMEM((H * 16,), jnp.float32),
            pltpu.VMEM((N,), jnp.float32),
            pltpu.VMEM((N,), jnp.float32),
            pltpu.VMEM((N,), jnp.float32),
            pltpu.VMEM((N,), jnp.float32),
            pltpu.VMEM((N,), jnp.float32),
        ],
    )


def _fin_body(x_ref, wr_ref, d0_ref, d1_ref, d2_ref, d3_ref, self_ref,
              projt_ref, c_ref, exp_ref, out_ref):
    dens = [jnp.sum(d[...], axis=0, keepdims=True)
            for d in (d0_ref, d1_ref, d2_ref, d3_ref)]
    den = jnp.concatenate(dens, axis=0)
    cnt = jnp.sum(self_ref[...], axis=0, keepdims=True)
    t = projt_ref[:H, :] + projt_ref[H:, :]
    e = jnp.maximum(t, t * 0.2)
    v = jnp.exp(e - c_ref[...])
    scale = jnp.where(cnt > 0.0, v / (den + 1e-16), 0.0)
    wide = lax.dot_general(
        scale, exp_ref[...],
        dimension_numbers=(((0,), (0,)), ((), ())),
        preferred_element_type=jnp.float32,
    )
    xt = jnp.dot(x_ref[...], wr_ref[...], preferred_element_type=jnp.float32)
    out_ref[...] = xt * wide


_fin_call = pl.pallas_call(
    _fin_body,
    grid=(NRB,),
    in_specs=[
        pl.BlockSpec((ROWB, IN_F), lambda r: (r, 0)),
        pl.BlockSpec((IN_F, H * OUT_F), lambda r: (0, 0)),
        pl.BlockSpec((NW, ROWB), lambda r: (0, r)),
        pl.BlockSpec((NW, ROWB), lambda r: (0, r)),
        pl.BlockSpec((NW, ROWB), lambda r: (0, r)),
        pl.BlockSpec((NW, ROWB), lambda r: (0, r)),
        pl.BlockSpec((NW, ROWB), lambda r: (0, r)),
        pl.BlockSpec((2 * H, ROWB), lambda r: (0, r)),
        pl.BlockSpec((H, 1), lambda r: (0, 0)),
        pl.BlockSpec((H, H * OUT_F), lambda r: (0, 0)),
    ],
    out_specs=pl.BlockSpec((ROWB, H * OUT_F), lambda r: (r, 0)),
    out_shape=jax.ShapeDtypeStruct((N, H * OUT_F), jnp.float32),
)


def kernel(x, edge_index, W, a):
    x = x.astype(jnp.float32)
    Wr = jnp.transpose(W, (1, 0, 2)).reshape(IN_F, H * OUT_F).astype(jnp.float32)
    a2 = a[..., 0].astype(jnp.float32)
    eye = jnp.eye(H, dtype=jnp.float32)
    ac_s = (a2[:, :OUT_F][:, :, None] * eye[:, None, :]).reshape(H * OUT_F, H)
    ac_d = (a2[:, OUT_F:][:, :, None] * eye[:, None, :]).reshape(H * OUT_F, H)
    Ac = jnp.concatenate([ac_s, ac_d], axis=1)

    projt, cmax = _proj_call(x, Wr, Ac)

    mub = cmax[0, :H] + cmax[0, H:]
    cbound = jnp.maximum(mub, 0.2 * mub)
    c_lanes = jnp.broadcast_to(cbound[:, None], (H, 16)).reshape(H * 16)

    edges = edge_index.astype(jnp.int32)
    d0, d1, d2, d3, self_parts = _make_edge_kernel()(projt, edges, c_lanes)

    expander = jnp.repeat(jnp.eye(H, dtype=jnp.float32), OUT_F, axis=1)
    out = _fin_call(
        x, Wr, d0, d1, d2, d3, self_parts,
        projt, cbound.reshape(H, 1), expander,
    )
    return out


# speedup vs baseline: 229.9457x; 1.0014x over previous
import functools

import jax
import jax.numpy as jnp
from jax import lax
from jax.experimental import pallas as pl
from jax.experimental.pallas import tpu as pltpu
from jax.experimental.pallas import tpu_sc as plsc

N = 4096
E = 131072
IN_F = 256
OUT_F = 64
H = 4
NW = 32
EPW = E // NW
ROWB = 512
NRB = N // ROWB


def _proj_body(x_ref, wr_ref, ac_ref, projt_ref, cmax_ref):
    r = pl.program_id(0)
    xt = jnp.dot(x_ref[...], wr_ref[...], preferred_element_type=jnp.float32)
    p = jnp.dot(xt, ac_ref[...], preferred_element_type=jnp.float32)
    projt_ref[...] = p.T
    pm = jnp.max(p, axis=0, keepdims=True)

    @pl.when(r == 0)
    def _():
        cmax_ref[...] = pm

    @pl.when(r > 0)
    def _():
        cmax_ref[...] = jnp.maximum(cmax_ref[...], pm)


_proj_call = pl.pallas_call(
    _proj_body,
    grid=(NRB,),
    in_specs=[
        pl.BlockSpec((ROWB, IN_F), lambda r: (r, 0)),
        pl.BlockSpec((IN_F, H * OUT_F), lambda r: (0, 0)),
        pl.BlockSpec((IN_F, 2 * H), lambda r: (0, 0)),
    ],
    out_specs=[
        pl.BlockSpec((2 * H, ROWB), lambda r: (0, r)),
        pl.BlockSpec((1, 2 * H), lambda r: (0, 0)),
    ],
    out_shape=[
        jax.ShapeDtypeStruct((2 * H, N), jnp.float32),
        jax.ShapeDtypeStruct((1, 2 * H), jnp.float32),
    ],
)


def _edge_kernel_body(projt_hbm, edge_hbm, c_hbm,
                      d0_out, d1_out, d2_out, d3_out, self_out,
                      projt_v, src_v, dst_v, c_v,
                      d0_v, d1_v, d2_v, d3_v, self_v):
    wid = lax.axis_index("s") * 2 + lax.axis_index("c")
    base = wid * EPW
    pltpu.sync_copy(projt_hbm, projt_v)
    pltpu.sync_copy(edge_hbm.at[0, pl.ds(base, EPW)], src_v)
    pltpu.sync_copy(edge_hbm.at[1, pl.ds(base, EPW)], dst_v)
    pltpu.sync_copy(c_hbm, c_v)

    den_refs = [d0_v, d1_v, d2_v, d3_v]
    zeros16 = jnp.zeros((16,), jnp.float32)
    ones16 = jnp.ones((16,), jnp.float32)

    @plsc.parallel_loop(0, N // 16, unroll=8)
    def _(i):
        sl = pl.ds(i * 16, 16)
        d0_v[sl] = zeros16
        d1_v[sl] = zeros16
        d2_v[sl] = zeros16
        d3_v[sl] = zeros16
        self_v[sl] = zeros16

    cvecs = [c_v[pl.ds(h * 16, 16)] for h in range(H)]
    rows = [jnp.full((16,), h, jnp.int32) for h in range(2 * H)]

    @plsc.parallel_loop(0, EPW // 16, unroll=8)
    def _(i):
        sl = pl.ds(i * 16, 16)
        s16 = src_v[sl]
        d16 = dst_v[sl]
        plsc.addupdate_scatter(
            self_v, [d16], jnp.where(s16 == d16, ones16, zeros16)
        )
        for h in range(H):
            gs = plsc.load_gather(projt_v, [rows[h], s16])
            gd = plsc.load_gather(projt_v, [rows[H + h], d16])
            t = gs + gd
            e = jnp.maximum(t, t * 0.2)
            v = jnp.exp(e - cvecs[h])
            plsc.addupdate_scatter(den_refs[h], [d16], v)

    pltpu.sync_copy(d0_v, d0_out.at[wid])
    pltpu.sync_copy(d1_v, d1_out.at[wid])
    pltpu.sync_copy(d2_v, d2_out.at[wid])
    pltpu.sync_copy(d3_v, d3_out.at[wid])
    pltpu.sync_copy(self_v, self_out.at[wid])


@functools.lru_cache(maxsize=1)
def _make_edge_kernel():
    mesh = plsc.VectorSubcoreMesh(core_axis_name="c", subcore_axis_name="s")
    return pl.kernel(
        _edge_kernel_body,
        out_type=[jax.ShapeDtypeStruct((NW, N), jnp.float32)] * 5,
        mesh=mesh,
        compiler_params=pltpu.CompilerParams(needs_layout_passes=False),
        scratch_types=[
            pltpu.VMEM((2 * H, N), jnp.float32),
            pltpu.VMEM((EPW,), jnp.int32),
            pltpu.VMEM((EPW,), jnp.int32),
            pltpu.VMEM((H * 16,), jnp.float32),
            pltpu.VMEM((N,), jnp.float32),
            pltpu.VMEM((N,), jnp.float32),
            pltpu.VMEM((N,), jnp.float32),
            pltpu.VMEM((N,), jnp.float32),
            pltpu.VMEM((N,), jnp.float32),
        ],
    )


def _fin_body(x_ref, wr_ref, d0_ref, d1_ref, d2_ref, d3_ref, self_ref,
              projt_ref, c_ref, exp_ref, out_ref):
    dens = [jnp.sum(d[...], axis=0, keepdims=True)
            for d in (d0_ref, d1_ref, d2_ref, d3_ref)]
    den = jnp.concatenate(dens, axis=0)
    cnt = jnp.sum(self_ref[...], axis=0, keepdims=True)
    t = projt_ref[:H, :] + projt_ref[H:, :]
    e = jnp.maximum(t, t * 0.2)
    v = jnp.exp(e - c_ref[...])
    scale = jnp.where(cnt > 0.0, v / (den + 1e-16), 0.0)
    wide = lax.dot_general(
        scale, exp_ref[...],
        dimension_numbers=(((0,), (0,)), ((), ())),
        preferred_element_type=jnp.float32,
    )
    xt = jnp.dot(x_ref[...], wr_ref[...], preferred_element_type=jnp.float32)
    out_ref[...] = xt * wide


_fin_call = pl.pallas_call(
    _fin_body,
    grid=(NRB,),
    in_specs=[
        pl.BlockSpec((ROWB, IN_F), lambda r: (r, 0)),
        pl.BlockSpec((IN_F, H * OUT_F), lambda r: (0, 0)),
        pl.BlockSpec((NW, ROWB), lambda r: (0, r)),
        pl.BlockSpec((NW, ROWB), lambda r: (0, r)),
        pl.BlockSpec((NW, ROWB), lambda r: (0, r)),
        pl.BlockSpec((NW, ROWB), lambda r: (0, r)),
        pl.BlockSpec((NW, ROWB), lambda r: (0, r)),
        pl.BlockSpec((2 * H, ROWB), lambda r: (0, r)),
        pl.BlockSpec((H, 1), lambda r: (0, 0)),
        pl.BlockSpec((H, H * OUT_F), lambda r: (0, 0)),
    ],
    out_specs=pl.BlockSpec((ROWB, H * OUT_F), lambda r: (r, 0)),
    out_shape=jax.ShapeDtypeStruct((N, H * OUT_F), jnp.float32),
)


def kernel(x, edge_index, W, a):
    x = x.astype(jnp.float32)
    Wr = jnp.transpose(W, (1, 0, 2)).reshape(IN_F, H * OUT_F).astype(jnp.float32)
    a2 = a[..., 0].astype(jnp.float32)
    eye = jnp.eye(H, dtype=jnp.float32)
    ac_s = (a2[:, :OUT_F][:, :, None] * eye[:, None, :]).reshape(H * OUT_F, H)
    ac_d = (a2[:, OUT_F:][:, :, None] * eye[:, None, :]).reshape(H * OUT_F, H)
    Ac = jnp.concatenate([ac_s, ac_d], axis=1)

    projt, cmax = _proj_call(x, Wr, Ac)

    mub = cmax[0, :H] + cmax[0, H:]
    cbound = jnp.maximum(mub, 0.2 * mub)
    c_lanes = jnp.broadcast_to(cbound[:, None], (H, 16)).reshape(H * 16)

    edges = edge_index.astype(jnp.int32)
    d0, d1, d2, d3, self_parts = _make_edge_kernel()(projt, edges, c_lanes)

    expander = jnp.repeat(jnp.eye(H, dtype=jnp.float32), OUT_F, axis=1)
    out = _fin_call(
        x, Wr, d0, d1, d2, d3, self_parts,
        projt, cbound.reshape(H, 1), expander,
    )
    return out


# speedup vs baseline: 274.5749x; 1.1941x over previous
import functools

import jax
import jax.numpy as jnp
from jax import lax
from jax.experimental import pallas as pl
from jax.experimental.pallas import tpu as pltpu
from jax.experimental.pallas import tpu_sc as plsc

N = 4096
E = 131072
IN_F = 256
OUT_F = 64
H = 4
NW = 32
EPW = E // NW
ROWB = 1024
NRB = N // ROWB


def _proj_body(x_ref, wr_ref, ac_ref, projt_ref, c_ref, cmax_ref):
    r = pl.program_id(0)
    xt = jnp.dot(x_ref[...], wr_ref[...], preferred_element_type=jnp.float32)
    p = jnp.dot(xt, ac_ref[...], preferred_element_type=jnp.float32)
    pt = p.T
    projt_ref[...] = pt
    pm = jnp.max(pt, axis=1, keepdims=True)

    @pl.when(r == 0)
    def _():
        cmax_ref[...] = pm

    @pl.when(r > 0)
    def _():
        cmax_ref[...] = jnp.maximum(cmax_ref[...], pm)

    @pl.when(r == NRB - 1)
    def _():
        cm = cmax_ref[...]
        mub = cm[:H, :] + cm[H:, :]
        cb = jnp.maximum(mub, mub * 0.2)
        c_ref[...] = jnp.broadcast_to(cb, (H, 16))


_proj_call = pl.pallas_call(
    _proj_body,
    grid=(NRB,),
    in_specs=[
        pl.BlockSpec((ROWB, IN_F), lambda r: (r, 0)),
        pl.BlockSpec((IN_F, H * OUT_F), lambda r: (0, 0)),
        pl.BlockSpec((IN_F, 2 * H), lambda r: (0, 0)),
    ],
    out_specs=[
        pl.BlockSpec((2 * H, ROWB), lambda r: (0, r)),
        pl.BlockSpec((H, 16), lambda r: (0, 0)),
    ],
    out_shape=[
        jax.ShapeDtypeStruct((2 * H, N), jnp.float32),
        jax.ShapeDtypeStruct((H, 16), jnp.float32),
    ],
    scratch_shapes=[pltpu.VMEM((2 * H, 1), jnp.float32)],
)


def _edge_kernel_body(projt_hbm, edge_hbm, c_hbm,
                      d0_out, d1_out, d2_out, d3_out, self_out,
                      projt_v, src_v, dst_v, c_v,
                      d0_v, d1_v, d2_v, d3_v, self_v):
    wid = lax.axis_index("s") * 2 + lax.axis_index("c")
    base = wid * EPW
    pltpu.sync_copy(projt_hbm, projt_v)
    pltpu.sync_copy(edge_hbm.at[0, pl.ds(base, EPW)], src_v)
    pltpu.sync_copy(edge_hbm.at[1, pl.ds(base, EPW)], dst_v)
    pltpu.sync_copy(c_hbm, c_v)

    den_refs = [d0_v, d1_v, d2_v, d3_v]
    zeros16 = jnp.zeros((16,), jnp.float32)
    ones16 = jnp.ones((16,), jnp.float32)

    @plsc.parallel_loop(0, N // 16, unroll=8)
    def _(i):
        sl = pl.ds(i * 16, 16)
        d0_v[sl] = zeros16
        d1_v[sl] = zeros16
        d2_v[sl] = zeros16
        d3_v[sl] = zeros16
        self_v[sl] = zeros16

    cvecs = [c_v[h, :] for h in range(H)]
    rows = [jnp.full((16,), h, jnp.int32) for h in range(2 * H)]

    @plsc.parallel_loop(0, EPW // 16, unroll=4)
    def _(i):
        sl = pl.ds(i * 16, 16)
        s16 = src_v[sl]
        d16 = dst_v[sl]
        plsc.addupdate_scatter(
            self_v, [d16], jnp.where(s16 == d16, ones16, zeros16)
        )
        for h in range(H):
            gs = plsc.load_gather(projt_v, [rows[h], s16])
            gd = plsc.load_gather(projt_v, [rows[H + h], d16])
            t = gs + gd
            e = jnp.maximum(t, t * 0.2)
            v = jnp.exp(e - cvecs[h])
            plsc.addupdate_scatter(den_refs[h], [d16], v)

    pltpu.sync_copy(d0_v, d0_out.at[wid])
    pltpu.sync_copy(d1_v, d1_out.at[wid])
    pltpu.sync_copy(d2_v, d2_out.at[wid])
    pltpu.sync_copy(d3_v, d3_out.at[wid])
    pltpu.sync_copy(self_v, self_out.at[wid])


@functools.lru_cache(maxsize=1)
def _make_edge_kernel():
    mesh = plsc.VectorSubcoreMesh(core_axis_name="c", subcore_axis_name="s")
    return pl.kernel(
        _edge_kernel_body,
        out_type=[jax.ShapeDtypeStruct((NW, N), jnp.float32)] * 5,
        mesh=mesh,
        compiler_params=pltpu.CompilerParams(needs_layout_passes=False),
        scratch_types=[
            pltpu.VMEM((2 * H, N), jnp.float32),
            pltpu.VMEM((EPW,), jnp.int32),
            pltpu.VMEM((EPW,), jnp.int32),
            pltpu.VMEM((H, 16), jnp.float32),
            pltpu.VMEM((N,), jnp.float32),
            pltpu.VMEM((N,), jnp.float32),
            pltpu.VMEM((N,), jnp.float32),
            pltpu.VMEM((N,), jnp.float32),
            pltpu.VMEM((N,), jnp.float32),
        ],
    )


def _fin_body(x_ref, wr_ref, d0_ref, d1_ref, d2_ref, d3_ref, self_ref,
              projt_ref, c_ref, exp_ref, out_ref):
    dens = [jnp.sum(d[...], axis=0, keepdims=True)
            for d in (d0_ref, d1_ref, d2_ref, d3_ref)]
    den = jnp.concatenate(dens, axis=0)
    cnt = jnp.sum(self_ref[...], axis=0, keepdims=True)
    t = projt_ref[:H, :] + projt_ref[H:, :]
    e = jnp.maximum(t, t * 0.2)
    v = jnp.exp(e - c_ref[:, 0:1])
    scale = jnp.where(cnt > 0.0, v / (den + 1e-16), 0.0)
    wide = lax.dot_general(
        scale, exp_ref[...],
        dimension_numbers=(((0,), (0,)), ((), ())),
        preferred_element_type=jnp.float32,
    )
    xt = jnp.dot(x_ref[...], wr_ref[...], preferred_element_type=jnp.float32)
    out_ref[...] = xt * wide


_fin_call = pl.pallas_call(
    _fin_body,
    grid=(NRB,),
    in_specs=[
        pl.BlockSpec((ROWB, IN_F), lambda r: (r, 0)),
        pl.BlockSpec((IN_F, H * OUT_F), lambda r: (0, 0)),
        pl.BlockSpec((NW, ROWB), lambda r: (0, r)),
        pl.BlockSpec((NW, ROWB), lambda r: (0, r)),
        pl.BlockSpec((NW, ROWB), lambda r: (0, r)),
        pl.BlockSpec((NW, ROWB), lambda r: (0, r)),
        pl.BlockSpec((NW, ROWB), lambda r: (0, r)),
        pl.BlockSpec((2 * H, ROWB), lambda r: (0, r)),
        pl.BlockSpec((H, 16), lambda r: (0, 0)),
        pl.BlockSpec((H, H * OUT_F), lambda r: (0, 0)),
    ],
    out_specs=pl.BlockSpec((ROWB, H * OUT_F), lambda r: (r, 0)),
    out_shape=jax.ShapeDtypeStruct((N, H * OUT_F), jnp.float32),
)


def kernel(x, edge_index, W, a):
    x = x.astype(jnp.float32)
    Wr = jnp.transpose(W, (1, 0, 2)).reshape(IN_F, H * OUT_F).astype(jnp.float32)
    a2 = a[..., 0].astype(jnp.float32)
    eye = jnp.eye(H, dtype=jnp.float32)
    ac_s = (a2[:, :OUT_F][:, :, None] * eye[:, None, :]).reshape(H * OUT_F, H)
    ac_d = (a2[:, OUT_F:][:, :, None] * eye[:, None, :]).reshape(H * OUT_F, H)
    Ac = jnp.concatenate([ac_s, ac_d], axis=1)

    projt, c_splat = _proj_call(x, Wr, Ac)

    edges = edge_index.astype(jnp.int32)
    d0, d1, d2, d3, self_parts = _make_edge_kernel()(projt, edges, c_splat)

    expander = jnp.repeat(jnp.eye(H, dtype=jnp.float32), OUT_F, axis=1)
    out = _fin_call(
        x, Wr, d0, d1, d2, d3, self_parts,
        projt, c_splat, expander,
    )
    return out
